# baseline (speedup 1.0000x reference)
import functools

import jax
import jax.numpy as jnp
from jax import lax
from jax.experimental import pallas as pl
from jax.experimental.pallas import tpu as pltpu
from jax.experimental.pallas import tpu_sc as plsc

NUM_LABELS = 1000000
EMBED_DIM = 32
INPUT_SIZE = 5
BATCH = 16384

_NC = 2
_NS = 16
_NW = _NC * _NS
_ROWS = BATCH * INPUT_SIZE
_RPW = _ROWS // _NW
_CHUNK = 128
_NCH = _RPW // _CHUNK
_PACK = 128 // EMBED_DIM
_PROWS = NUM_LABELS // _PACK


def _sc_gather(q3, table128, wm):
    mesh = plsc.VectorSubcoreMesh(core_axis_name="c", subcore_axis_name="s")

    @functools.partial(
        pl.kernel,
        mesh=mesh,
        out_type=jax.ShapeDtypeStruct((_ROWS, EMBED_DIM), jnp.float32),
        scratch_types=[
            pltpu.VMEM((_NCH, _CHUNK), jnp.int32),
            pltpu.VMEM((_CHUNK, 128), jnp.float32),
            pltpu.VMEM((_CHUNK, 128), jnp.float32),
            pltpu.VMEM((_CHUNK, EMBED_DIM), jnp.float32),
            pltpu.SemaphoreType.DMA,
        ],
    )
    def k(q_hbm, tab_hbm, wm_hbm, out_hbm, q_v, gbuf, m_v, obuf, sem):
        wid = lax.axis_index("s") * _NC + lax.axis_index("c")
        base = wid * _RPW

        pltpu.sync_copy(q_hbm.at[wid], q_v)

        for c in range(_NCH):
            cp = pltpu.async_copy(tab_hbm.at[q_v.at[c]], gbuf, sem)
            pltpu.sync_copy(
                wm_hbm.at[pl.ds(base + c * _CHUNK, _CHUNK)], m_v
            )
            cp.wait()

            def body(j, carry):
                acc0 = gbuf[j, pl.ds(0, 16)] * m_v[j, pl.ds(0, 16)]
                acc1 = gbuf[j, pl.ds(16, 16)] * m_v[j, pl.ds(16, 16)]
                for r in range(1, _PACK):
                    acc0 = acc0 + gbuf[j, pl.ds(r * 32, 16)] * m_v[j, pl.ds(r * 32, 16)]
                    acc1 = acc1 + gbuf[j, pl.ds(r * 32 + 16, 16)] * m_v[j, pl.ds(r * 32 + 16, 16)]
                obuf[j, pl.ds(0, 16)] = acc0
                obuf[j, pl.ds(16, 16)] = acc1
                return carry

            lax.fori_loop(0, _CHUNK, body, 0)

            pltpu.sync_copy(obuf, out_hbm.at[pl.ds(base + c * _CHUNK, _CHUNK)])

    return k(q3, table128, wm)


def kernel(x, table, weight):
    xi = x.astype(jnp.int32).reshape(-1)
    q3 = (xi // _PACK).reshape(_NW, _NCH, _CHUNK)
    table128 = table.reshape(_PROWS, _PACK * EMBED_DIM)
    lane_q = jnp.arange(128, dtype=jnp.int32) // EMBED_DIM
    sel = (lane_q[None, :] == (xi % _PACK)[:, None]).astype(jnp.float32)
    wrow = jnp.tile(weight.astype(jnp.float32), BATCH)
    wm = sel * wrow[:, None]
    out = _sc_gather(q3, table128, wm)
    return out.reshape(BATCH, INPUT_SIZE * EMBED_DIM)


# speedup vs baseline: 1.0522x; 1.0522x over previous
import functools

import jax
import jax.numpy as jnp
from jax import lax
from jax.experimental import pallas as pl
from jax.experimental.pallas import tpu as pltpu
from jax.experimental.pallas import tpu_sc as plsc

NUM_LABELS = 1000000
EMBED_DIM = 32
INPUT_SIZE = 5
BATCH = 16384

_NC = 2
_NS = 16
_NW = _NC * _NS
_ROWS = BATCH * INPUT_SIZE
_RPW = _ROWS // _NW
_CHUNK = 128
_NCH = _RPW // _CHUNK
_PACK = 128 // EMBED_DIM
_PROWS = NUM_LABELS // _PACK


def _sc_gather(q3, table128, wm):
    mesh = plsc.VectorSubcoreMesh(core_axis_name="c", subcore_axis_name="s")

    @functools.partial(
        pl.kernel,
        mesh=mesh,
        out_type=jax.ShapeDtypeStruct((_ROWS, EMBED_DIM), jnp.float32),
        scratch_types=[
            pltpu.VMEM((_NCH, _CHUNK), jnp.int32),
            pltpu.VMEM((2, _CHUNK, 128), jnp.float32),
            pltpu.VMEM((2, _CHUNK, 128), jnp.float32),
            pltpu.VMEM((2, _CHUNK, EMBED_DIM), jnp.float32),
            pltpu.SemaphoreType.DMA,
            pltpu.SemaphoreType.DMA,
            pltpu.SemaphoreType.DMA,
        ],
    )
    def k(q_hbm, tab_hbm, wm_hbm, out_hbm, q_v, gbuf, m_v, obuf, sg, sm, so):
        wid = lax.axis_index("s") * _NC + lax.axis_index("c")
        base = wid * _RPW

        pltpu.sync_copy(q_hbm.at[wid], q_v)

        def fire(c):
            s = c % 2
            g = pltpu.async_copy(tab_hbm.at[q_v.at[c]], gbuf.at[s], sg)
            m = pltpu.async_copy(
                wm_hbm.at[pl.ds(base + c * _CHUNK, _CHUNK)], m_v.at[s], sm
            )
            return g, m

        pend = fire(0)
        outs = []
        for c in range(_NCH):
            s = c % 2
            nxt = fire(c + 1) if c + 1 < _NCH else None
            pend[0].wait()
            pend[1].wait()
            if c >= 2:
                outs[c - 2].wait()

            def body(j, carry):
                acc0 = gbuf[s, j, pl.ds(0, 16)] * m_v[s, j, pl.ds(0, 16)]
                acc1 = gbuf[s, j, pl.ds(16, 16)] * m_v[s, j, pl.ds(16, 16)]
                for r in range(1, _PACK):
                    acc0 = acc0 + gbuf[s, j, pl.ds(r * 32, 16)] * m_v[s, j, pl.ds(r * 32, 16)]
                    acc1 = acc1 + gbuf[s, j, pl.ds(r * 32 + 16, 16)] * m_v[s, j, pl.ds(r * 32 + 16, 16)]
                obuf[s, j, pl.ds(0, 16)] = acc0
                obuf[s, j, pl.ds(16, 16)] = acc1
                return carry

            lax.fori_loop(0, _CHUNK, body, 0)

            outs.append(
                pltpu.async_copy(
                    obuf.at[s], out_hbm.at[pl.ds(base + c * _CHUNK, _CHUNK)], so
                )
            )
            pend = nxt

        for cp in outs[-2:]:
            cp.wait()

    return k(q3, table128, wm)


def kernel(x, table, weight):
    xi = x.astype(jnp.int32).reshape(-1)
    q3 = (xi // _PACK).reshape(_NW, _NCH, _CHUNK)
    table128 = table.reshape(_PROWS, _PACK * EMBED_DIM)
    lane_q = jnp.arange(128, dtype=jnp.int32) // EMBED_DIM
    sel = (lane_q[None, :] == (xi % _PACK)[:, None]).astype(jnp.float32)
    wrow = jnp.tile(weight.astype(jnp.float32), BATCH)
    wm = sel * wrow[:, None]
    out = _sc_gather(q3, table128, wm)
    return out.reshape(BATCH, INPUT_SIZE * EMBED_DIM)
